# baseline (speedup 1.0000x reference)
import functools

import jax
import jax.numpy as jnp
from jax import lax
from jax.experimental import pallas as pl
from jax.experimental.pallas import tpu as pltpu
from jax.experimental.pallas import tpu_sc as plsc

N = 10000
D = 128
E = 320000

NC = 2
NS = 16
CHUNK = 128
CPW0 = 114
CPW1 = 45
NCHUNKS = NS * (CPW0 + CPW1)
EP = NCHUNKS * CHUNK
ROWS_PER_TILE = 632
NPAD = NS * ROWS_PER_TILE
NBUF = 3


def _sc_agg_body(h_hbm, ei_hbm, zero_hbm, out_hbm,
                 sidx, didx, rows, acc, s0, s1, s2):
    c = lax.axis_index("c")
    s = lax.axis_index("s")
    sems = (s0, s1, s2)

    r0 = s * ROWS_PER_TILE
    pltpu.sync_copy(zero_hbm, acc.at[pl.ds(r0, ROWS_PER_TILE)])
    plsc.subcore_barrier()

    cpw = CPW1 + c * (CPW0 - CPW1)
    groups = (CPW1 // NBUF) + c * ((CPW0 - CPW1) // NBUF)
    cbase = (1 - c) * (NS * CPW0) + s * cpw

    def idx_start(j, b):
        cj = cbase + j
        pltpu.async_copy(ei_hbm.at[0, cj], sidx.at[b], sems[b])
        pltpu.async_copy(ei_hbm.at[1, cj], didx.at[b], sems[b])

    def idx_wait(j, b):
        cj = cbase + j
        pltpu.make_async_copy(ei_hbm.at[0, cj], sidx.at[b], sems[b]).wait()
        pltpu.make_async_copy(ei_hbm.at[1, cj], didx.at[b], sems[b]).wait()

    def gather_start(b):
        pltpu.async_copy(h_hbm.at[sidx.at[b]], rows.at[b], sems[b])

    def gather_wait(b):
        pltpu.make_async_copy(h_hbm.at[sidx.at[b]], rows.at[b],
                              sems[b]).wait()

    def scat_start(b):
        pltpu.async_copy(rows.at[b], acc.at[didx.at[b]], sems[b], add=True)

    def scat_wait(b):
        pltpu.make_async_copy(rows.at[b], acc.at[didx.at[b]], sems[b]).wait()

    idx_start(0, 0)
    idx_start(1, 1)
    idx_wait(0, 0)
    gather_start(0)

    def group(g, carry):
        @pl.when(g < groups)
        def _():
            for b in range(NBUF):
                i = g * NBUF + b
                b2 = (b + 2) % NBUF
                b1 = (b + 1) % NBUF

                if b == 0:
                    @pl.when(g > 0)
                    def _():
                        scat_wait(b2)
                else:
                    scat_wait(b2)

                @pl.when(i + 2 < cpw)
                def _():
                    idx_start(i + 2, b2)

                @pl.when(i + 1 < cpw)
                def _():
                    idx_wait(i + 1, b1)
                    gather_start(b1)

                gather_wait(b)
                scat_start(b)
        return carry

    lax.fori_loop(0, CPW0 // NBUF, group, 0)
    scat_wait(2)
    plsc.subcore_barrier()

    pltpu.sync_copy(acc.at[pl.ds(r0, ROWS_PER_TILE)],
                    out_hbm.at[c, pl.ds(r0, ROWS_PER_TILE)])


_sc_agg = functools.partial(
    pl.kernel,
    out_type=jax.ShapeDtypeStruct((NC, NPAD, D), jnp.float32),
    mesh=plsc.VectorSubcoreMesh(
        core_axis_name="c", subcore_axis_name="s",
        num_cores=NC, num_subcores=NS),
    scratch_types=[
        pltpu.VMEM((NBUF, CHUNK), jnp.int32),
        pltpu.VMEM((NBUF, CHUNK), jnp.int32),
        pltpu.VMEM((NBUF, CHUNK, D), jnp.float32),
        pltpu.VMEM_SHARED((NPAD, D), jnp.float32),
        pltpu.SemaphoreType.DMA,
        pltpu.SemaphoreType.DMA,
        pltpu.SemaphoreType.DMA,
    ],
)(_sc_agg_body)


BLK = 1000


def _mlp_body(p_ref, h_ref, wa_ref, ba_ref, wb_ref, bb_ref, o_ref):
    z = p_ref[0] + p_ref[1] + h_ref[...]
    z = jnp.dot(z, wa_ref[...], preferred_element_type=jnp.float32)
    z = jnp.maximum(z + ba_ref[...], 0.0)
    z = jnp.dot(z, wb_ref[...], preferred_element_type=jnp.float32)
    o_ref[...] = z + bb_ref[...]


def _mlp(parts, h, Wa, ba, Wb, bb):
    grid = (N + BLK - 1) // BLK
    return pl.pallas_call(
        _mlp_body,
        grid=(grid,),
        in_specs=[
            pl.BlockSpec((NC, BLK, D), lambda i: (0, i, 0)),
            pl.BlockSpec((BLK, D), lambda i: (i, 0)),
            pl.BlockSpec((D, D), lambda i: (0, 0)),
            pl.BlockSpec((1, D), lambda i: (0, 0)),
            pl.BlockSpec((D, D), lambda i: (0, 0)),
            pl.BlockSpec((1, D), lambda i: (0, 0)),
        ],
        out_specs=pl.BlockSpec((BLK, D), lambda i: (i, 0)),
        out_shape=jax.ShapeDtypeStruct((N, D), jnp.float32),
    )(parts, h, Wa, ba.reshape(1, D), Wb, bb.reshape(1, D))


def kernel(x, edge_index, W0a, b0a, W0b, b0b, W1a, b1a, W1b, b1b,
           W2a, b2a, W2b, b2b):
    ei = edge_index.astype(jnp.int32)
    pad = jnp.stack([jnp.zeros((EP - E,), jnp.int32),
                     N + (jnp.arange(EP - E, dtype=jnp.int32) % (NPAD - N))])
    ei3 = jnp.concatenate([ei, pad], axis=1).reshape(2, NCHUNKS, CHUNK)
    zeros = jnp.zeros((ROWS_PER_TILE, D), jnp.float32)
    h = x
    for Wa, ba, Wb, bb in ((W0a, b0a, W0b, b0b),
                           (W1a, b1a, W1b, b1b),
                           (W2a, b2a, W2b, b2b)):
        parts = _sc_agg(h, ei3, zeros)
        h = _mlp(parts, h, Wa, ba, Wb, bb)
    return h


# speedup vs baseline: 2.8152x; 2.8152x over previous
import functools

import jax
import jax.numpy as jnp
from jax import lax
from jax.experimental import pallas as pl
from jax.experimental.pallas import tpu as pltpu
from jax.experimental.pallas import tpu_sc as plsc

N = 10000
D = 128
E = 320000

NC = 2
NS = 16
CHUNK = 128
NW = NC * NS
CPW = 81
NCHUNKS = NW * CPW
EP = NCHUNKS * CHUNK
ROWS_PER_TILE = 632
NPAD = NS * ROWS_PER_TILE
NBUF = 3


def _sc_agg_body(h_hbm, ei_hbm, zero_hbm, out_hbm,
                 sidx, didx, rows, acc, s0, s1, s2):
    c = lax.axis_index("c")
    s = lax.axis_index("s")
    sems = (s0, s1, s2)

    r0 = s * ROWS_PER_TILE
    pltpu.sync_copy(zero_hbm, acc.at[pl.ds(r0, ROWS_PER_TILE)])
    plsc.subcore_barrier()

    cbase = (c * NS + s) * CPW

    def idx_start(j, b):
        cj = cbase + j
        pltpu.async_copy(ei_hbm.at[0, cj], sidx.at[b], sems[b])
        pltpu.async_copy(ei_hbm.at[1, cj], didx.at[b], sems[b])

    def idx_wait(j, b):
        cj = cbase + j
        pltpu.make_async_copy(ei_hbm.at[0, cj], sidx.at[b], sems[b]).wait()
        pltpu.make_async_copy(ei_hbm.at[1, cj], didx.at[b], sems[b]).wait()

    def gather_start(b):
        pltpu.async_copy(h_hbm.at[sidx.at[b]], rows.at[b], sems[b])

    def gather_wait(b):
        pltpu.make_async_copy(h_hbm.at[sidx.at[b]], rows.at[b],
                              sems[b]).wait()

    def scat_start(b):
        pltpu.async_copy(rows.at[b], acc.at[didx.at[b]], sems[b], add=True)

    def scat_wait(b):
        pltpu.make_async_copy(rows.at[b], acc.at[didx.at[b]], sems[b]).wait()

    idx_start(0, 0)
    idx_start(1, 1)
    idx_wait(0, 0)
    gather_start(0)

    def group(g, carry):
        for b in range(NBUF):
            i = g * NBUF + b
            b2 = (b + 2) % NBUF
            b1 = (b + 1) % NBUF

            if b == 0:
                @pl.when(g > 0)
                def _():
                    scat_wait(b2)
            else:
                scat_wait(b2)

            @pl.when(i + 2 < CPW)
            def _():
                idx_start(i + 2, b2)

            @pl.when(i + 1 < CPW)
            def _():
                idx_wait(i + 1, b1)
                gather_start(b1)

            gather_wait(b)
            scat_start(b)
        return carry

    lax.fori_loop(0, CPW // NBUF, group, 0)
    scat_wait(2)
    plsc.subcore_barrier()

    pltpu.sync_copy(acc.at[pl.ds(r0, ROWS_PER_TILE)],
                    out_hbm.at[c, pl.ds(r0, ROWS_PER_TILE)])


_sc_agg = functools.partial(
    pl.kernel,
    out_type=jax.ShapeDtypeStruct((NC, NPAD, D), jnp.float32),
    mesh=plsc.VectorSubcoreMesh(
        core_axis_name="c", subcore_axis_name="s",
        num_cores=NC, num_subcores=NS),
    scratch_types=[
        pltpu.VMEM((NBUF, CHUNK), jnp.int32),
        pltpu.VMEM((NBUF, CHUNK), jnp.int32),
        pltpu.VMEM((NBUF, CHUNK, D), jnp.float32),
        pltpu.VMEM_SHARED((NPAD, D), jnp.float32),
        pltpu.SemaphoreType.DMA,
        pltpu.SemaphoreType.DMA,
        pltpu.SemaphoreType.DMA,
    ],
)(_sc_agg_body)


BLK = 1000


def _mlp_body(p_ref, h_ref, wa_ref, ba_ref, wb_ref, bb_ref, o_ref):
    z = p_ref[0] + p_ref[1] + h_ref[...]
    z = jnp.dot(z, wa_ref[...], preferred_element_type=jnp.float32)
    z = jnp.maximum(z + ba_ref[...], 0.0)
    z = jnp.dot(z, wb_ref[...], preferred_element_type=jnp.float32)
    o_ref[...] = z + bb_ref[...]


def _mlp(parts, h, Wa, ba, Wb, bb):
    grid = (N + BLK - 1) // BLK
    return pl.pallas_call(
        _mlp_body,
        grid=(grid,),
        in_specs=[
            pl.BlockSpec((NC, BLK, D), lambda i: (0, i, 0)),
            pl.BlockSpec((BLK, D), lambda i: (i, 0)),
            pl.BlockSpec((D, D), lambda i: (0, 0)),
            pl.BlockSpec((1, D), lambda i: (0, 0)),
            pl.BlockSpec((D, D), lambda i: (0, 0)),
            pl.BlockSpec((1, D), lambda i: (0, 0)),
        ],
        out_specs=pl.BlockSpec((BLK, D), lambda i: (i, 0)),
        out_shape=jax.ShapeDtypeStruct((N, D), jnp.float32),
    )(parts, h, Wa, ba.reshape(1, D), Wb, bb.reshape(1, D))


def kernel(x, edge_index, W0a, b0a, W0b, b0b, W1a, b1a, W1b, b1b,
           W2a, b2a, W2b, b2b):
    ei = edge_index.astype(jnp.int32)
    padn = jnp.arange(EP - E, dtype=jnp.int32)
    pad = jnp.stack([padn % N, N + padn % (NPAD - N)])
    ei3 = jnp.concatenate([ei, pad], axis=1).reshape(2, NCHUNKS, CHUNK)
    zeros = jnp.zeros((ROWS_PER_TILE, D), jnp.float32)
    h = x
    for Wa, ba, Wb, bb in ((W0a, b0a, W0b, b0b),
                           (W1a, b1a, W1b, b1b),
                           (W2a, b2a, W2b, b2b)):
        parts = _sc_agg(h, ei3, zeros)
        h = _mlp(parts, h, Wa, ba, Wb, bb)
    return h


# speedup vs baseline: 2.8248x; 1.0034x over previous
import functools

import jax
import jax.numpy as jnp
from jax import lax
from jax.experimental import pallas as pl
from jax.experimental.pallas import tpu as pltpu
from jax.experimental.pallas import tpu_sc as plsc

N = 10000
D = 128
E = 320000

NC = 2
NS = 16
CHUNK = 128
NW = NC * NS
NCHUNKS = E // CHUNK
CPW = NCHUNKS // NW
TAIL = NCHUNKS - NW * CPW
ROWS_PER_TILE = 632
NPAD = NS * ROWS_PER_TILE
NBUF = 3


def _sc_agg_body(h_hbm, ei_hbm, zero_hbm, out_hbm,
                 sidx, didx, rows, acc, s0, s1, s2):
    c = lax.axis_index("c")
    s = lax.axis_index("s")
    sems = (s0, s1, s2)

    r0 = s * ROWS_PER_TILE
    pltpu.sync_copy(zero_hbm, acc.at[pl.ds(r0, ROWS_PER_TILE)])
    plsc.subcore_barrier()

    cbase = (c * NS + s) * CPW

    def idx_start(cj, b):
        pltpu.async_copy(ei_hbm.at[0, cj], sidx.at[b], sems[b])
        pltpu.async_copy(ei_hbm.at[1, cj], didx.at[b], sems[b])

    def idx_wait(cj, b):
        pltpu.make_async_copy(ei_hbm.at[0, cj], sidx.at[b], sems[b]).wait()
        pltpu.make_async_copy(ei_hbm.at[1, cj], didx.at[b], sems[b]).wait()

    def gather_start(b):
        pltpu.async_copy(h_hbm.at[sidx.at[b]], rows.at[b], sems[b])

    def gather_wait(b):
        pltpu.make_async_copy(h_hbm.at[sidx.at[b]], rows.at[b],
                              sems[b]).wait()

    def scat_start(b):
        pltpu.async_copy(rows.at[b], acc.at[didx.at[b]], sems[b], add=True)

    def scat_wait(b):
        pltpu.make_async_copy(rows.at[b], acc.at[didx.at[b]], sems[b]).wait()

    idx_start(cbase, 0)
    idx_start(cbase + 1, 1)
    idx_wait(cbase, 0)
    gather_start(0)

    def group(g, carry):
        for b in range(NBUF):
            i = g * NBUF + b
            b2 = (b + 2) % NBUF
            b1 = (b + 1) % NBUF

            if b == 0:
                @pl.when(g > 0)
                def _():
                    scat_wait(b2)
            else:
                scat_wait(b2)

            @pl.when(i + 2 < CPW)
            def _():
                idx_start(cbase + i + 2, b2)

            @pl.when(i + 1 < CPW)
            def _():
                idx_wait(cbase + i + 1, b1)
                gather_start(b1)

            gather_wait(b)
            scat_start(b)
        return carry

    lax.fori_loop(0, CPW // NBUF, group, 0)
    scat_wait(2)

    @pl.when(c * NS + s < TAIL)
    def _():
        cj = NW * CPW + c * NS + s
        idx_start(cj, 0)
        idx_wait(cj, 0)
        gather_start(0)
        gather_wait(0)
        scat_start(0)
        scat_wait(0)

    plsc.subcore_barrier()

    pltpu.sync_copy(acc.at[pl.ds(r0, ROWS_PER_TILE)],
                    out_hbm.at[c, pl.ds(r0, ROWS_PER_TILE)])


_sc_agg = functools.partial(
    pl.kernel,
    out_type=jax.ShapeDtypeStruct((NC, NPAD, D), jnp.float32),
    mesh=plsc.VectorSubcoreMesh(
        core_axis_name="c", subcore_axis_name="s",
        num_cores=NC, num_subcores=NS),
    scratch_types=[
        pltpu.VMEM((NBUF, CHUNK), jnp.int32),
        pltpu.VMEM((NBUF, CHUNK), jnp.int32),
        pltpu.VMEM((NBUF, CHUNK, D), jnp.float32),
        pltpu.VMEM_SHARED((NPAD, D), jnp.float32),
        pltpu.SemaphoreType.DMA,
        pltpu.SemaphoreType.DMA,
        pltpu.SemaphoreType.DMA,
    ],
)(_sc_agg_body)


BLK = 1000


def _mlp_body(p_ref, h_ref, wa_ref, ba_ref, wb_ref, bb_ref, o_ref):
    z = p_ref[0] + p_ref[1] + h_ref[...]
    z = jnp.dot(z, wa_ref[...], preferred_element_type=jnp.float32)
    z = jnp.maximum(z + ba_ref[...], 0.0)
    z = jnp.dot(z, wb_ref[...], preferred_element_type=jnp.float32)
    o_ref[...] = z + bb_ref[...]


def _mlp(parts, h, Wa, ba, Wb, bb):
    grid = (N + BLK - 1) // BLK
    return pl.pallas_call(
        _mlp_body,
        grid=(grid,),
        in_specs=[
            pl.BlockSpec((NC, BLK, D), lambda i: (0, i, 0)),
            pl.BlockSpec((BLK, D), lambda i: (i, 0)),
            pl.BlockSpec((D, D), lambda i: (0, 0)),
            pl.BlockSpec((1, D), lambda i: (0, 0)),
            pl.BlockSpec((D, D), lambda i: (0, 0)),
            pl.BlockSpec((1, D), lambda i: (0, 0)),
        ],
        out_specs=pl.BlockSpec((BLK, D), lambda i: (i, 0)),
        out_shape=jax.ShapeDtypeStruct((N, D), jnp.float32),
    )(parts, h, Wa, ba.reshape(1, D), Wb, bb.reshape(1, D))


def kernel(x, edge_index, W0a, b0a, W0b, b0b, W1a, b1a, W1b, b1b,
           W2a, b2a, W2b, b2b):
    ei3 = edge_index.astype(jnp.int32).reshape(2, NCHUNKS, CHUNK)
    zeros = jnp.zeros((ROWS_PER_TILE, D), jnp.float32)
    h = x
    for Wa, ba, Wb, bb in ((W0a, b0a, W0b, b0b),
                           (W1a, b1a, W1b, b1b),
                           (W2a, b2a, W2b, b2b)):
        parts = _sc_agg(h, ei3, zeros)
        h = _mlp(parts, h, Wa, ba, Wb, bb)
    return h


# speedup vs baseline: 2.8883x; 1.0225x over previous
import functools

import jax
import jax.numpy as jnp
from jax import lax
from jax.experimental import pallas as pl
from jax.experimental.pallas import tpu as pltpu
from jax.experimental.pallas import tpu_sc as plsc

N = 10000
D = 128
E = 320000

NC = 2
NS = 16
CHUNK = 128
NW = NC * NS
NCHUNKS = E // CHUNK
CPW = NCHUNKS // NW
TAIL = NCHUNKS - NW * CPW
ROWS_PER_TILE = 632
NPAD = NS * ROWS_PER_TILE
NBUF = 3


def _sc_agg_body(h_hbm, ei_hbm, zero_hbm, out_hbm,
                 sidx, didx, rows, acc, s0, s1, s2):
    c = lax.axis_index("c")
    s = lax.axis_index("s")
    sems = (s0, s1, s2)

    r0 = s * ROWS_PER_TILE
    pltpu.sync_copy(zero_hbm, acc.at[pl.ds(r0, ROWS_PER_TILE)])
    plsc.subcore_barrier()

    cbase = (c * NS + s) * CPW

    def idx_start(cj, b):
        pltpu.async_copy(ei_hbm.at[0, cj], sidx.at[b], sems[b])
        pltpu.async_copy(ei_hbm.at[1, cj], didx.at[b], sems[b])

    def idx_wait(cj, b):
        pltpu.make_async_copy(ei_hbm.at[0, cj], sidx.at[b], sems[b]).wait()
        pltpu.make_async_copy(ei_hbm.at[1, cj], didx.at[b], sems[b]).wait()

    def gather_start(b):
        pltpu.async_copy(h_hbm.at[sidx.at[b]], rows.at[b], sems[b])

    def gather_wait(b):
        pltpu.make_async_copy(h_hbm.at[sidx.at[b]], rows.at[b],
                              sems[b]).wait()

    def scat_start(b):
        pltpu.async_copy(rows.at[b], acc.at[didx.at[b]], sems[b], add=True)

    def scat_wait(b):
        pltpu.make_async_copy(rows.at[b], acc.at[didx.at[b]], sems[b]).wait()

    idx_start(cbase, 0)
    idx_start(cbase + 1, 1)
    idx_wait(cbase, 0)
    gather_start(0)

    def group(g, carry):
        for b in range(NBUF):
            i = g * NBUF + b
            b2 = (b + 2) % NBUF
            b1 = (b + 1) % NBUF

            if b == 0:
                @pl.when(g > 0)
                def _():
                    scat_wait(b2)
            else:
                scat_wait(b2)

            @pl.when(i + 2 < CPW)
            def _():
                idx_start(cbase + i + 2, b2)

            @pl.when(i + 1 < CPW)
            def _():
                idx_wait(cbase + i + 1, b1)
                gather_start(b1)

            gather_wait(b)
            scat_start(b)
        return carry

    lax.fori_loop(0, CPW // NBUF, group, 0)
    scat_wait(2)

    @pl.when(c * NS + s < TAIL)
    def _():
        cj = NW * CPW + c * NS + s
        idx_start(cj, 0)
        idx_wait(cj, 0)
        gather_start(0)
        gather_wait(0)
        scat_start(0)
        scat_wait(0)

    plsc.subcore_barrier()

    pltpu.sync_copy(acc.at[pl.ds(r0, ROWS_PER_TILE)],
                    out_hbm.at[c, pl.ds(r0, ROWS_PER_TILE)])


_sc_agg = functools.partial(
    pl.kernel,
    out_type=jax.ShapeDtypeStruct((NC, NPAD, D), jnp.float32),
    mesh=plsc.VectorSubcoreMesh(
        core_axis_name="c", subcore_axis_name="s",
        num_cores=NC, num_subcores=NS),
    scratch_types=[
        pltpu.VMEM((NBUF, CHUNK), jnp.int32),
        pltpu.VMEM((NBUF, CHUNK), jnp.int32),
        pltpu.VMEM((NBUF, CHUNK, D), jnp.float32),
        pltpu.VMEM_SHARED((NPAD, D), jnp.float32),
        pltpu.SemaphoreType.DMA,
        pltpu.SemaphoreType.DMA,
        pltpu.SemaphoreType.DMA,
    ],
)(_sc_agg_body)


BLK = 2000


def _mlp_body(p_ref, h_ref, wa_ref, ba_ref, wb_ref, bb_ref, o_ref):
    z = p_ref[0] + p_ref[1] + h_ref[...]
    z = jnp.dot(z, wa_ref[...], preferred_element_type=jnp.float32)
    z = jnp.maximum(z + ba_ref[...], 0.0)
    z = jnp.dot(z, wb_ref[...], preferred_element_type=jnp.float32)
    o_ref[...] = z + bb_ref[...]


def _mlp(parts, h, Wa, ba, Wb, bb):
    grid = (N + BLK - 1) // BLK
    return pl.pallas_call(
        _mlp_body,
        grid=(grid,),
        in_specs=[
            pl.BlockSpec((NC, BLK, D), lambda i: (0, i, 0)),
            pl.BlockSpec((BLK, D), lambda i: (i, 0)),
            pl.BlockSpec((D, D), lambda i: (0, 0)),
            pl.BlockSpec((1, D), lambda i: (0, 0)),
            pl.BlockSpec((D, D), lambda i: (0, 0)),
            pl.BlockSpec((1, D), lambda i: (0, 0)),
        ],
        out_specs=pl.BlockSpec((BLK, D), lambda i: (i, 0)),
        out_shape=jax.ShapeDtypeStruct((N, D), jnp.float32),
    )(parts, h, Wa, ba.reshape(1, D), Wb, bb.reshape(1, D))


def kernel(x, edge_index, W0a, b0a, W0b, b0b, W1a, b1a, W1b, b1b,
           W2a, b2a, W2b, b2b):
    ei3 = edge_index.astype(jnp.int32).reshape(2, NCHUNKS, CHUNK)
    zeros = jnp.zeros((ROWS_PER_TILE, D), jnp.float32)
    h = x
    for Wa, ba, Wb, bb in ((W0a, b0a, W0b, b0b),
                           (W1a, b1a, W1b, b1b),
                           (W2a, b2a, W2b, b2b)):
        parts = _sc_agg(h, ei3, zeros)
        h = _mlp(parts, h, Wa, ba, Wb, bb)
    return h


# speedup vs baseline: 2.9583x; 1.0242x over previous
import functools

import jax
import jax.numpy as jnp
from jax import lax
from jax.experimental import pallas as pl
from jax.experimental.pallas import tpu as pltpu
from jax.experimental.pallas import tpu_sc as plsc

N = 10000
D = 128
E = 320000

NC = 2
NS = 16
CHUNK = 128
NW = NC * NS
NCHUNKS = E // CHUNK
CPW = NCHUNKS // NW
TAIL = NCHUNKS - NW * CPW
ROWS_PER_TILE = 632
NPAD = NS * ROWS_PER_TILE
NBUF = 3


def _sc_agg_body(h_hbm, ei_hbm, zero_hbm, out_hbm,
                 sidx, didx, rows, acc, s0, s1, s2):
    c = lax.axis_index("c")
    s = lax.axis_index("s")
    sems = (s0, s1, s2)

    r0 = s * ROWS_PER_TILE
    pltpu.sync_copy(zero_hbm, acc.at[pl.ds(r0, ROWS_PER_TILE)])
    plsc.subcore_barrier()

    cbase = (c * NS + s) * CPW

    def idx_start(cj, b):
        e0 = cj * CHUNK
        pltpu.async_copy(ei_hbm.at[0, pl.ds(e0, CHUNK)], sidx.at[b], sems[b])
        pltpu.async_copy(ei_hbm.at[1, pl.ds(e0, CHUNK)], didx.at[b], sems[b])

    def idx_wait(cj, b):
        e0 = cj * CHUNK
        pltpu.make_async_copy(ei_hbm.at[0, pl.ds(e0, CHUNK)], sidx.at[b],
                              sems[b]).wait()
        pltpu.make_async_copy(ei_hbm.at[1, pl.ds(e0, CHUNK)], didx.at[b],
                              sems[b]).wait()

    def gather_start(b):
        pltpu.async_copy(h_hbm.at[sidx.at[b]], rows.at[b], sems[b])

    def gather_wait(b):
        pltpu.make_async_copy(h_hbm.at[sidx.at[b]], rows.at[b],
                              sems[b]).wait()

    def scat_start(b):
        pltpu.async_copy(rows.at[b], acc.at[didx.at[b]], sems[b], add=True)

    def scat_wait(b):
        pltpu.make_async_copy(rows.at[b], acc.at[didx.at[b]], sems[b]).wait()

    idx_start(cbase, 0)
    idx_start(cbase + 1, 1)
    idx_wait(cbase, 0)
    gather_start(0)

    def group(g, carry):
        for b in range(NBUF):
            i = g * NBUF + b
            b2 = (b + 2) % NBUF
            b1 = (b + 1) % NBUF

            if b == 0:
                @pl.when(g > 0)
                def _():
                    scat_wait(b2)
            else:
                scat_wait(b2)

            @pl.when(i + 2 < CPW)
            def _():
                idx_start(cbase + i + 2, b2)

            @pl.when(i + 1 < CPW)
            def _():
                idx_wait(cbase + i + 1, b1)
                gather_start(b1)

            gather_wait(b)
            scat_start(b)
        return carry

    lax.fori_loop(0, CPW // NBUF, group, 0)
    scat_wait(2)

    @pl.when(c * NS + s < TAIL)
    def _():
        cj = NW * CPW + c * NS + s
        idx_start(cj, 0)
        idx_wait(cj, 0)
        gather_start(0)
        gather_wait(0)
        scat_start(0)
        scat_wait(0)

    plsc.subcore_barrier()

    pltpu.sync_copy(acc.at[pl.ds(r0, ROWS_PER_TILE)],
                    out_hbm.at[c, pl.ds(r0, ROWS_PER_TILE)])


_sc_agg = functools.partial(
    pl.kernel,
    out_type=jax.ShapeDtypeStruct((NC, NPAD, D), jnp.float32),
    mesh=plsc.VectorSubcoreMesh(
        core_axis_name="c", subcore_axis_name="s",
        num_cores=NC, num_subcores=NS),
    scratch_types=[
        pltpu.VMEM((NBUF, CHUNK), jnp.int32),
        pltpu.VMEM((NBUF, CHUNK), jnp.int32),
        pltpu.VMEM((NBUF, CHUNK, D), jnp.float32),
        pltpu.VMEM_SHARED((NPAD, D), jnp.float32),
        pltpu.SemaphoreType.DMA,
        pltpu.SemaphoreType.DMA,
        pltpu.SemaphoreType.DMA,
    ],
)(_sc_agg_body)


BLK = 2000


def _mlp_body(p_ref, h_ref, wa_ref, ba_ref, wb_ref, bb_ref, o_ref):
    z = p_ref[0] + p_ref[1] + h_ref[...]
    z = jnp.dot(z, wa_ref[...], preferred_element_type=jnp.float32)
    z = jnp.maximum(z + ba_ref[...], 0.0)
    z = jnp.dot(z, wb_ref[...], preferred_element_type=jnp.float32)
    o_ref[...] = z + bb_ref[...]


def _mlp(parts, h, Wa, ba, Wb, bb):
    grid = (N + BLK - 1) // BLK
    return pl.pallas_call(
        _mlp_body,
        grid=(grid,),
        in_specs=[
            pl.BlockSpec((NC, BLK, D), lambda i: (0, i, 0)),
            pl.BlockSpec((BLK, D), lambda i: (i, 0)),
            pl.BlockSpec((D, D), lambda i: (0, 0)),
            pl.BlockSpec((1, D), lambda i: (0, 0)),
            pl.BlockSpec((D, D), lambda i: (0, 0)),
            pl.BlockSpec((1, D), lambda i: (0, 0)),
        ],
        out_specs=pl.BlockSpec((BLK, D), lambda i: (i, 0)),
        out_shape=jax.ShapeDtypeStruct((N, D), jnp.float32),
    )(parts, h, Wa, ba.reshape(1, D), Wb, bb.reshape(1, D))


def kernel(x, edge_index, W0a, b0a, W0b, b0b, W1a, b1a, W1b, b1b,
           W2a, b2a, W2b, b2b):
    ei = edge_index.astype(jnp.int32)
    zeros = jnp.zeros((ROWS_PER_TILE, D), jnp.float32)
    h = x
    for Wa, ba, Wb, bb in ((W0a, b0a, W0b, b0b),
                           (W1a, b1a, W1b, b1b),
                           (W2a, b2a, W2b, b2b)):
        parts = _sc_agg(h, ei, zeros)
        h = _mlp(parts, h, Wa, ba, Wb, bb)
    return h
